# Initial kernel scaffold; baseline (speedup 1.0000x reference)
import jax
import jax.numpy as jnp
from jax.experimental import pallas as pl


def kernel(features1, features2, x1, x2, nuv1, nuv2, topk, W1, b1, W2, b2):
    raise NotImplementedError("write your pallas kernel here")


# speedup vs baseline: 1.8709x; 1.8709x over previous
import functools

import jax
import jax.numpy as jnp
from jax import lax
from jax.experimental import pallas as pl
from jax.experimental.pallas import tpu as pltpu
from jax.experimental.pallas import tpu_sc as plsc

N1 = 10000
N2 = 10000
NN = 32
NI = 128
ND = 128
N1R = 10240
L = 16

_PREP_BM = 1000


def _prep_body(f2_ref, w1a_ref, f1_ref, w1b_ref, b1_ref, g_ref, s_ref):
    g_ref[...] = jnp.dot(f2_ref[...], w1a_ref[...],
                         preferred_element_type=jnp.float32)
    s_ref[...] = (
        jnp.dot(f1_ref[...], w1b_ref[...], preferred_element_type=jnp.float32)
        + b1_ref[...]
    )


def _prep(features2, w1a, features1, w1b, b1r):
    return pl.pallas_call(
        _prep_body,
        grid=(N2 // _PREP_BM,),
        in_specs=[
            pl.BlockSpec((_PREP_BM, NI), lambda j: (j, 0)),
            pl.BlockSpec((NI, ND), lambda j: (0, 0)),
            pl.BlockSpec((_PREP_BM, NI), lambda j: (j, 0)),
            pl.BlockSpec((NI, ND), lambda j: (0, 0)),
            pl.BlockSpec((1, ND), lambda j: (0, 0)),
        ],
        out_specs=[
            pl.BlockSpec((_PREP_BM, ND), lambda j: (j, 0)),
            pl.BlockSpec((_PREP_BM, ND), lambda j: (j, 0)),
        ],
        out_shape=[
            jax.ShapeDtypeStruct((N2, ND), jnp.float32),
            jax.ShapeDtypeStruct((N1, ND), jnp.float32),
        ],
    )(features2, w1a, features1, w1b, b1r)


_CH = 80


def _sc_gather(table, idx_flat, x2cs, x1cs):
    info = plsc.get_sparse_core_info()
    nw = info.num_cores * info.num_subcores
    epw = (N1R * NN) // nw
    nch = epw // _CH

    mesh = plsc.VectorSubcoreMesh(core_axis_name="c", subcore_axis_name="s")

    @functools.partial(
        pl.kernel,
        mesh=mesh,
        out_type=[
            jax.ShapeDtypeStruct((N1R * NN, ND), jnp.float32),
            jax.ShapeDtypeStruct((N1R * NN,), jnp.float32),
        ],
        scratch_types=[
            pltpu.VMEM((_CH,), jnp.int32),
            pltpu.VMEM((_CH, ND), jnp.float32),
            pltpu.VMEM((_CH,), jnp.float32),
            pltpu.VMEM((N2,), jnp.float32),
            pltpu.VMEM((N2,), jnp.float32),
            pltpu.VMEM((N2,), jnp.float32),
            pltpu.VMEM((N1R,), jnp.float32),
            pltpu.VMEM((N1R,), jnp.float32),
            pltpu.VMEM((N1R,), jnp.float32),
            pltpu.SemaphoreType.DMA,
        ],
        compiler_params=pltpu.CompilerParams(needs_layout_passes=False),
    )
    def k(t_hbm, idx_hbm, x2x_hbm, x2y_hbm, x2z_hbm, x1x_hbm, x1y_hbm,
          x1z_hbm, r_hbm, dis_hbm,
          idx_v, rows_v, disb_v, x2x, x2y, x2z, x1x, x1y, x1z, sem):
        wid = lax.axis_index("s") * info.num_cores + lax.axis_index("c")
        base = wid * epw
        pltpu.sync_copy(x2x_hbm, x2x)
        pltpu.sync_copy(x2y_hbm, x2y)
        pltpu.sync_copy(x2z_hbm, x2z)
        pltpu.sync_copy(x1x_hbm, x1x)
        pltpu.sync_copy(x1y_hbm, x1y)
        pltpu.sync_copy(x1z_hbm, x1z)

        def body(c, carry):
            off = base + c * _CH
            pltpu.sync_copy(idx_hbm.at[pl.ds(off, _CH)], idx_v)
            cp = pltpu.async_copy(t_hbm.at[idx_v], rows_v, sem)
            for g in range(_CH // L):
                iv = idx_v[pl.ds(g * L, L)]
                gx = plsc.load_gather(x2x, [iv])
                gy = plsc.load_gather(x2y, [iv])
                gz = plsc.load_gather(x2z, [iv])
                ivec = lax.shift_right_logical(
                    off + g * L + lax.iota(jnp.int32, L), 5)
                sx = plsc.load_gather(x1x, [ivec])
                sy = plsc.load_gather(x1y, [ivec])
                sz = plsc.load_gather(x1z, [ivec])
                dx = gx - sx
                dy = gy - sy
                dz = gz - sz
                d2 = dx * dx + dy * dy + dz * dz
                w = jnp.exp(-0.5 * d2)
                disb_v[pl.ds(g * L, L)] = jnp.where(iv == 0, 0.0, w)
            cp.wait()
            pltpu.sync_copy(rows_v, r_hbm.at[pl.ds(off, _CH)])
            pltpu.sync_copy(disb_v, dis_hbm.at[pl.ds(off, _CH)])
            return carry

        lax.fori_loop(0, nch, body, 0)

    return k(table, idx_flat, *x2cs, *x1cs)


_BM = 400


def _elu(x):
    return jnp.where(x > 0, x, jnp.exp(jnp.minimum(x, 0.0)) - 1.0)


def _main_body(r_ref, dis_ref, s_ref, w2_ref, b2_ref, o_ref):
    s = s_ref[...]
    w2 = w2_ref[...]
    b2 = b2_ref[...]
    acc = jnp.zeros((_BM, ND), jnp.float32)
    for n in range(NN):
        h1 = _elu(r_ref[:, n * ND:(n + 1) * ND] + s)
        h2 = _elu(jnp.dot(h1, w2, preferred_element_type=jnp.float32) + b2)
        acc = acc + dis_ref[:, n:n + 1] * h2
    o_ref[...] = acc


def _main(r2, dis2, s, w2, b2r):
    return pl.pallas_call(
        _main_body,
        grid=(N1 // _BM,),
        in_specs=[
            pl.BlockSpec((_BM, NN * ND), lambda j: (j, 0)),
            pl.BlockSpec((_BM, NN), lambda j: (j, 0)),
            pl.BlockSpec((_BM, ND), lambda j: (j, 0)),
            pl.BlockSpec((ND, ND), lambda j: (0, 0)),
            pl.BlockSpec((1, ND), lambda j: (0, 0)),
        ],
        out_specs=pl.BlockSpec((_BM, ND), lambda j: (j, 0)),
        out_shape=jax.ShapeDtypeStruct((N1, ND), jnp.float32),
    )(r2, dis2, s, w2, b2r)


def kernel(features1, features2, x1, x2, nuv1, nuv2, topk, W1, b1, W2, b2):
    w1a = W1[:NI]
    w1b = W1[NI:]
    b1r = b1.reshape(1, ND)
    b2r = b2.reshape(1, ND)

    idx_flat = jnp.pad(topk, ((0, N1R - N1), (0, 0))).reshape(N1R * NN)
    x2cs = [x2[:, c] for c in range(3)]
    x1p = jnp.pad(x1, ((0, N1R - N1), (0, 0)))
    x1cs = [x1p[:, c] for c in range(3)]

    g_table, s = _prep(features2, w1a, features1, w1b, b1r)
    r, dis = _sc_gather(g_table, idx_flat, x2cs, x1cs)
    r2 = r.reshape(N1R, NN * ND)
    dis2 = dis.reshape(N1R, NN)
    return _main(r2, dis2, s, W2, b2r)


# speedup vs baseline: 2.2852x; 1.2214x over previous
import functools

import jax
import jax.numpy as jnp
from jax import lax
from jax.experimental import pallas as pl
from jax.experimental.pallas import tpu as pltpu
from jax.experimental.pallas import tpu_sc as plsc

N1 = 10000
N2 = 10000
NN = 32
NI = 128
ND = 128
N1R = 10240
L = 16

_PREP_BM = 1000


def _prep_body(f2_ref, w1a_ref, f1_ref, w1b_ref, b1_ref, g_ref, s_ref):
    g_ref[...] = jnp.dot(f2_ref[...], w1a_ref[...],
                         preferred_element_type=jnp.float32)
    s_ref[...] = (
        jnp.dot(f1_ref[...], w1b_ref[...], preferred_element_type=jnp.float32)
        + b1_ref[...]
    )


def _prep(features2, w1a, features1, w1b, b1r):
    return pl.pallas_call(
        _prep_body,
        grid=(N2 // _PREP_BM,),
        in_specs=[
            pl.BlockSpec((_PREP_BM, NI), lambda j: (j, 0)),
            pl.BlockSpec((NI, ND), lambda j: (0, 0)),
            pl.BlockSpec((_PREP_BM, NI), lambda j: (j, 0)),
            pl.BlockSpec((NI, ND), lambda j: (0, 0)),
            pl.BlockSpec((1, ND), lambda j: (0, 0)),
        ],
        out_specs=[
            pl.BlockSpec((_PREP_BM, ND), lambda j: (j, 0)),
            pl.BlockSpec((_PREP_BM, ND), lambda j: (j, 0)),
        ],
        out_shape=[
            jax.ShapeDtypeStruct((N2, ND), jnp.float32),
            jax.ShapeDtypeStruct((N1, ND), jnp.float32),
        ],
    )(features2, w1a, features1, w1b, b1r)


_CH = 128
_NBUF = 4


def _sc_gather(table, idx_flat, x2cs, x1cs):
    info = plsc.get_sparse_core_info()
    nw = info.num_cores * info.num_subcores
    epw = (N1R * NN) // nw
    ipw = N1R // nw
    nch = epw // _CH
    ngrp = nch // _NBUF

    mesh = plsc.VectorSubcoreMesh(core_axis_name="c", subcore_axis_name="s")

    @functools.partial(
        pl.kernel,
        mesh=mesh,
        out_type=[
            jax.ShapeDtypeStruct((N1R * NN, ND), jnp.float32),
            jax.ShapeDtypeStruct((N1R * NN,), jnp.float32),
        ],
        scratch_types=[
            pltpu.VMEM((epw,), jnp.int32),
            pltpu.VMEM((epw,), jnp.float32),
            [pltpu.VMEM((_CH, ND), jnp.float32)] * _NBUF,
            pltpu.VMEM((N2,), jnp.float32),
            pltpu.VMEM((N2,), jnp.float32),
            pltpu.VMEM((N2,), jnp.float32),
            pltpu.VMEM((ipw,), jnp.float32),
            pltpu.VMEM((ipw,), jnp.float32),
            pltpu.VMEM((ipw,), jnp.float32),
            [pltpu.SemaphoreType.DMA] * _NBUF,
            [pltpu.SemaphoreType.DMA] * _NBUF,
            pltpu.SemaphoreType.DMA,
        ],
        compiler_params=pltpu.CompilerParams(needs_layout_passes=False),
    )
    def k(t_hbm, idx_hbm, x2x_hbm, x2y_hbm, x2z_hbm, x1x_hbm, x1y_hbm,
          x1z_hbm, r_hbm, dis_hbm,
          idx_v, disb_v, rows, x2x, x2y, x2z, x1x, x1y, x1z,
          gsem, wsem, psem):
        wid = lax.axis_index("s") * info.num_cores + lax.axis_index("c")
        base = wid * epw

        pltpu.async_copy(idx_hbm.at[pl.ds(base, epw)], idx_v, psem)
        pltpu.async_copy(x2x_hbm, x2x, psem)
        pltpu.async_copy(x2y_hbm, x2y, psem)
        pltpu.async_copy(x2z_hbm, x2z, psem)
        pltpu.async_copy(x1x_hbm.at[pl.ds(wid * ipw, ipw)], x1x, psem)
        pltpu.async_copy(x1y_hbm.at[pl.ds(wid * ipw, ipw)], x1y, psem)
        pltpu.async_copy(x1z_hbm.at[pl.ds(wid * ipw, ipw)], x1z, psem)
        pltpu.make_async_copy(idx_hbm.at[pl.ds(base, epw)], idx_v, psem).wait()
        pltpu.make_async_copy(x2x_hbm, x2x, psem).wait()
        pltpu.make_async_copy(x2y_hbm, x2y, psem).wait()
        pltpu.make_async_copy(x2z_hbm, x2z, psem).wait()
        pltpu.make_async_copy(x1x_hbm.at[pl.ds(0, ipw)], x1x, psem).wait()
        pltpu.make_async_copy(x1y_hbm.at[pl.ds(0, ipw)], x1y, psem).wait()
        pltpu.make_async_copy(x1z_hbm.at[pl.ds(0, ipw)], x1z, psem).wait()

        def _fire_gather(c, b):
            pltpu.async_copy(
                t_hbm.at[idx_v.at[pl.ds(c * _CH, _CH)]], rows[b], gsem[b])

        def _wait_gather(b):
            pltpu.make_async_copy(
                t_hbm.at[idx_v.at[pl.ds(0, _CH)]], rows[b], gsem[b]).wait()

        def _fire_write(c, b):
            pltpu.async_copy(
                rows[b], r_hbm.at[pl.ds(base + c * _CH, _CH)], wsem[b])

        def _wait_write(b):
            pltpu.make_async_copy(
                rows[b], r_hbm.at[pl.ds(base, _CH)], wsem[b]).wait()

        for b in range(_NBUF):
            _fire_gather(b, b)

        def body(g, carry):
            for b in range(_NBUF):
                c = g * _NBUF + b
                off = c * _CH
                for gi in range(_CH // L):
                    iv = idx_v[pl.ds(off + gi * L, L)]
                    gx = plsc.load_gather(x2x, [iv])
                    gy = plsc.load_gather(x2y, [iv])
                    gz = plsc.load_gather(x2z, [iv])
                    ivec = lax.shift_right_logical(
                        off + gi * L + lax.iota(jnp.int32, L), 5)
                    sx = plsc.load_gather(x1x, [ivec])
                    sy = plsc.load_gather(x1y, [ivec])
                    sz = plsc.load_gather(x1z, [ivec])
                    dx = gx - sx
                    dy = gy - sy
                    dz = gz - sz
                    d2 = dx * dx + dy * dy + dz * dz
                    w = jnp.exp(-0.5 * d2)
                    disb_v[pl.ds(off + gi * L, L)] = jnp.where(iv == 0, 0.0, w)
                _wait_gather(b)
                _fire_write(c, b)
                _wait_write(b)
                @pl.when(g < ngrp - 1)
                def _():
                    _fire_gather(c + _NBUF, b)
            return carry

        lax.fori_loop(0, ngrp, body, 0)
        pltpu.sync_copy(disb_v, dis_hbm.at[pl.ds(base, epw)])

    return k(table, idx_flat, *x2cs, *x1cs)


_BM = 400


def _elu(x):
    return jnp.where(x > 0, x, jnp.exp(jnp.minimum(x, 0.0)) - 1.0)


def _main_body(r_ref, dis_ref, s_ref, w2_ref, b2_ref, o_ref):
    s = s_ref[...]
    w2 = w2_ref[...]
    b2 = b2_ref[...]
    acc = jnp.zeros((_BM, ND), jnp.float32)
    for n in range(NN):
        h1 = _elu(r_ref[:, n * ND:(n + 1) * ND] + s)
        h2 = _elu(jnp.dot(h1, w2, preferred_element_type=jnp.float32) + b2)
        acc = acc + dis_ref[:, n:n + 1] * h2
    o_ref[...] = acc


def _main(r2, dis2, s, w2, b2r):
    return pl.pallas_call(
        _main_body,
        grid=(N1 // _BM,),
        in_specs=[
            pl.BlockSpec((_BM, NN * ND), lambda j: (j, 0)),
            pl.BlockSpec((_BM, NN), lambda j: (j, 0)),
            pl.BlockSpec((_BM, ND), lambda j: (j, 0)),
            pl.BlockSpec((ND, ND), lambda j: (0, 0)),
            pl.BlockSpec((1, ND), lambda j: (0, 0)),
        ],
        out_specs=pl.BlockSpec((_BM, ND), lambda j: (j, 0)),
        out_shape=jax.ShapeDtypeStruct((N1, ND), jnp.float32),
    )(r2, dis2, s, w2, b2r)


def kernel(features1, features2, x1, x2, nuv1, nuv2, topk, W1, b1, W2, b2):
    w1a = W1[:NI]
    w1b = W1[NI:]
    b1r = b1.reshape(1, ND)
    b2r = b2.reshape(1, ND)

    idx_flat = jnp.pad(topk, ((0, N1R - N1), (0, 0))).reshape(N1R * NN)
    x2cs = [x2[:, c] for c in range(3)]
    x1p = jnp.pad(x1, ((0, N1R - N1), (0, 0)))
    x1cs = [x1p[:, c] for c in range(3)]

    g_table, s = _prep(features2, w1a, features1, w1b, b1r)
    r, dis = _sc_gather(g_table, idx_flat, x2cs, x1cs)
    r2 = r.reshape(N1R, NN * ND)
    dis2 = dis.reshape(N1R, NN)
    return _main(r2, dis2, s, W2, b2r)
